# baseline (speedup 1.0000x reference)
import functools

import jax
import jax.numpy as jnp
from jax import lax
from jax.experimental import pallas as pl
from jax.experimental.pallas import tpu as pltpu
from jax.experimental.pallas import tpu_sc as plsc

BATCH = 16384
EMBED_DIM = 32
OUT_DIM = 32

_NC = 2
_NS = 16
_NW = _NC * _NS
_B_PER_W = BATCH // _NW
_CHUNK = 128
_NCHUNK = _B_PER_W // _CHUNK


def _sc_gather_body(idx_hbm, par_hbm, code_hbm, hier_hbm, ec_out, ep_out,
                    idx_v, par_v, rows_c, rows_p, sem_c, sem_p):
    wid = lax.axis_index("s") * _NC + lax.axis_index("c")
    base = wid * _B_PER_W
    for j in range(_NCHUNK):
        pltpu.sync_copy(idx_hbm.at[pl.ds(base + j * _CHUNK, _CHUNK)], idx_v.at[j])
        pltpu.sync_copy(par_hbm.at[pl.ds(base + j * _CHUNK, _CHUNK)], par_v.at[j])
    copies = []
    for j in range(_NCHUNK):
        copies.append(pltpu.async_copy(
            code_hbm.at[idx_v.at[j]], rows_c.at[pl.ds(j * _CHUNK, _CHUNK)],
            sem_c))
        copies.append(pltpu.async_copy(
            hier_hbm.at[par_v.at[j]], rows_p.at[pl.ds(j * _CHUNK, _CHUNK)],
            sem_p))
    for cp in copies:
        cp.wait()
    pltpu.sync_copy(rows_c, ec_out.at[pl.ds(base, _B_PER_W)])
    pltpu.sync_copy(rows_p, ep_out.at[pl.ds(base, _B_PER_W)])


_sc_gather = functools.partial(
    pl.kernel,
    out_type=[
        jax.ShapeDtypeStruct((BATCH, EMBED_DIM), jnp.float32),
        jax.ShapeDtypeStruct((BATCH, EMBED_DIM), jnp.float32),
    ],
    mesh=plsc.VectorSubcoreMesh(core_axis_name="c", subcore_axis_name="s"),
    compiler_params=pltpu.CompilerParams(use_tc_tiling_on_sc=False),
    scratch_types=[
        pltpu.VMEM((_NCHUNK, _CHUNK), jnp.int32),
        pltpu.VMEM((_NCHUNK, _CHUNK), jnp.int32),
        pltpu.VMEM((_B_PER_W, EMBED_DIM), jnp.float32),
        pltpu.VMEM((_B_PER_W, EMBED_DIM), jnp.float32),
        pltpu.SemaphoreType.DMA,
        pltpu.SemaphoreType.DMA,
    ],
)(_sc_gather_body)


_MM_BLK = 2048


def _mm_body(ec_ref, ep_ref, w1_ref, w2_ref, b_ref, o_ref):
    acc = jnp.dot(ec_ref[...], w1_ref[...], preferred_element_type=jnp.float32)
    acc += jnp.dot(ep_ref[...], w2_ref[...], preferred_element_type=jnp.float32)
    o_ref[...] = acc + b_ref[...]


def _tc_project(ec, ep, w1, w2, b2d):
    grid = (BATCH // _MM_BLK,)
    return pl.pallas_call(
        _mm_body,
        grid=grid,
        in_specs=[
            pl.BlockSpec((_MM_BLK, EMBED_DIM), lambda i: (i, 0)),
            pl.BlockSpec((_MM_BLK, EMBED_DIM), lambda i: (i, 0)),
            pl.BlockSpec((EMBED_DIM, OUT_DIM), lambda i: (0, 0)),
            pl.BlockSpec((EMBED_DIM, OUT_DIM), lambda i: (0, 0)),
            pl.BlockSpec((1, OUT_DIM), lambda i: (0, 0)),
        ],
        out_specs=pl.BlockSpec((_MM_BLK, OUT_DIM), lambda i: (i, 0)),
        out_shape=jax.ShapeDtypeStruct((BATCH, OUT_DIM), jnp.float32),
    )(ec, ep, w1, w2, b2d)


@jax.jit
def kernel(indices, parents, code_table, hier_table, W, b):
    ec, ep = _sc_gather(indices, parents, code_table, hier_table)
    w1 = W[:EMBED_DIM]
    w2 = W[EMBED_DIM:]
    return _tc_project(ec, ep, w1, w2, b.reshape(1, OUT_DIM))


# speedup vs baseline: 1.0759x; 1.0759x over previous
import functools

import jax
import jax.numpy as jnp
from jax import lax
from jax.experimental import pallas as pl
from jax.experimental.pallas import tpu as pltpu
from jax.experimental.pallas import tpu_sc as plsc

BATCH = 16384
VOCAB = 100000
HIER_VOCAB = 10000
EMBED_DIM = 32
OUT_DIM = 32

_NC = 2
_NS = 16
_NW = _NC * _NS
_B_PER_W = BATCH // _NW
_CHUNK = 128
_NCHUNK = _B_PER_W // _CHUNK


def _sc_gather_body(idx_hbm, par_hbm, code_hbm, hier_hbm, ec_out, ep_out,
                    idx_v, par_v, rows_c, rows_p, sem_c, sem_p):
    wid = lax.axis_index("s") * _NC + lax.axis_index("c")
    base = wid * _B_PER_W
    for j in range(_NCHUNK):
        pltpu.sync_copy(idx_hbm.at[pl.ds(base + j * _CHUNK, _CHUNK)], idx_v.at[j])
        pltpu.sync_copy(par_hbm.at[pl.ds(base + j * _CHUNK, _CHUNK)], par_v.at[j])
    copies = []
    for j in range(_NCHUNK):
        copies.append(pltpu.async_copy(
            code_hbm.at[idx_v.at[j]], rows_c.at[pl.ds(j * _CHUNK, _CHUNK)],
            sem_c))
        copies.append(pltpu.async_copy(
            hier_hbm.at[par_v.at[j]], rows_p.at[pl.ds(j * _CHUNK, _CHUNK)],
            sem_p))
    for cp in copies:
        cp.wait()
    pltpu.sync_copy(rows_c, ec_out.at[pl.ds(base, _B_PER_W)])
    pltpu.sync_copy(rows_p, ep_out.at[pl.ds(base, _B_PER_W)])


_sc_gather = functools.partial(
    pl.kernel,
    out_type=[
        jax.ShapeDtypeStruct((BATCH, EMBED_DIM), jnp.float32),
        jax.ShapeDtypeStruct((BATCH, EMBED_DIM), jnp.float32),
    ],
    mesh=plsc.VectorSubcoreMesh(core_axis_name="c", subcore_axis_name="s"),
    compiler_params=pltpu.CompilerParams(use_tc_tiling_on_sc=False),
    scratch_types=[
        pltpu.VMEM((_NCHUNK, _CHUNK), jnp.int32),
        pltpu.VMEM((_NCHUNK, _CHUNK), jnp.int32),
        pltpu.VMEM((_B_PER_W, EMBED_DIM), jnp.float32),
        pltpu.VMEM((_B_PER_W, EMBED_DIM), jnp.float32),
        pltpu.SemaphoreType.DMA,
        pltpu.SemaphoreType.DMA,
    ],
)(_sc_gather_body)


_MM_BLK = 2048


def _mm_body(ec_ref, ep_ref, w1_ref, w2_ref, b_ref, o_ref):
    acc = jax.lax.dot_general(
        w1_ref[...], ec_ref[...], (((0,), (1,)), ((), ())),
        preferred_element_type=jnp.float32)
    acc += jax.lax.dot_general(
        w2_ref[...], ep_ref[...], (((0,), (1,)), ((), ())),
        preferred_element_type=jnp.float32)
    o_ref[...] = acc + b_ref[...]


def _tc_project(ec, ep, w1, w2, bcol):
    grid = (BATCH // _MM_BLK,)
    return pl.pallas_call(
        _mm_body,
        grid=grid,
        in_specs=[
            pl.BlockSpec((_MM_BLK, EMBED_DIM), lambda i: (i, 0)),
            pl.BlockSpec((_MM_BLK, EMBED_DIM), lambda i: (i, 0)),
            pl.BlockSpec((EMBED_DIM, OUT_DIM), lambda i: (0, 0)),
            pl.BlockSpec((EMBED_DIM, OUT_DIM), lambda i: (0, 0)),
            pl.BlockSpec((OUT_DIM, 1), lambda i: (0, 0)),
        ],
        out_specs=pl.BlockSpec((OUT_DIM, _MM_BLK), lambda i: (0, i)),
        out_shape=jax.ShapeDtypeStruct((OUT_DIM, BATCH), jnp.float32),
    )(ec, ep, w1, w2, bcol)


@jax.jit
def kernel(indices, parents, code_table, hier_table, W, b):
    ec, ep = _sc_gather(indices, parents, code_table, hier_table)
    w1 = W[:EMBED_DIM]
    w2 = W[EMBED_DIM:]
    out_t = _tc_project(ec, ep, w1, w2, b.reshape(OUT_DIM, 1))
    return out_t.T
